# Initial kernel scaffold; baseline (speedup 1.0000x reference)
import jax
import jax.numpy as jnp
from jax.experimental import pallas as pl


def kernel(x, edge_index, W1, a_src1, a_dst1, b1, gamma, beta, W2, a_src2, a_dst2, b2):
    raise NotImplementedError("write your pallas kernel here")


# speedup vs baseline: 22.9392x; 22.9392x over previous
import functools
import jax
import jax.numpy as jnp
from jax import lax
from jax.experimental import pallas as pl
from jax.experimental.pallas import tpu as pltpu
from jax.experimental.pallas import tpu_sc as plsc

N = 10000
E = 320000
D = 128
DA = 144
ET = E + N
NW = 32
EB = 128
NBLK = 81
CHUNK = EB * NBLK
ET_PAD = NW * CHUNK
NP = 10240
RPT = NP // 16


def _ones_col(n):
    return jnp.where(lax.broadcasted_iota(jnp.int32, (n, 16), 1) == 0, 1.0, 0.0)


def _dense_pre(x_ref, w_ref, a_ref, haug_ref, asad_ref):
    h = jnp.dot(x_ref[...], w_ref[...], preferred_element_type=jnp.float32)
    haug_ref[:, :D] = h
    haug_ref[:, D:DA] = _ones_col(N)
    asad_ref[...] = jnp.dot(h, a_ref[...], preferred_element_type=jnp.float32)


def _dense_mid(op_ref, b_ref, g_ref, be_ref, w_ref, a_ref, haug_ref, asad_ref):
    num = op_ref[0, :N, :D] + op_ref[1, :N, :D]
    denc = op_ref[0, :N, D:DA] + op_ref[1, :N, D:DA]
    den = denc[:, 0:1]
    h = num / (den + 1e-16) + b_ref[...]
    mu = jnp.mean(h, axis=0, keepdims=True)
    var = jnp.mean(h * h, axis=0, keepdims=True) - mu * mu
    h = (h - mu) / jnp.sqrt(var + 1e-5) * g_ref[...] + be_ref[...]
    h = jnp.maximum(h, 0.0)
    h2 = jnp.dot(h, w_ref[...], preferred_element_type=jnp.float32)
    haug_ref[:, :D] = h2
    haug_ref[:, D:DA] = _ones_col(N)
    asad_ref[...] = jnp.dot(h2, a_ref[...], preferred_element_type=jnp.float32)


def _dense_final(op_ref, b_ref, out_ref):
    num = op_ref[0, :N, :D] + op_ref[1, :N, :D]
    denc = op_ref[0, :N, D:DA] + op_ref[1, :N, D:DA]
    out_ref[...] = num / (denc[:, 0:1] + 1e-16) + b_ref[...]


_SC_MESH = plsc.VectorSubcoreMesh(core_axis_name="c", subcore_axis_name="s")
_SC_PARAMS = pltpu.CompilerParams(needs_layout_passes=False,
                                  use_tc_tiling_on_sc=False)
SB = 27


@functools.partial(
    pl.kernel,
    out_type=jax.ShapeDtypeStruct((NW, CHUNK), jnp.float32),
    mesh=_SC_MESH,
    compiler_params=_SC_PARAMS,
    scratch_types=[
        pltpu.VMEM((2 * N,), jnp.float32),
        pltpu.VMEM((NBLK, EB), jnp.int32),
        pltpu.VMEM((NBLK, EB), jnp.int32),
        pltpu.VMEM((CHUNK,), jnp.float32),
    ],
)
def _sc_weights(asad_hbm, src_hbm, dst_hbm, w_hbm, tab_v, src_v, dst_v, w_v):
    c = lax.axis_index("c")
    s = lax.axis_index("s")
    wid = s * 2 + c

    pltpu.sync_copy(asad_hbm, tab_v)
    pltpu.sync_copy(src_hbm.at[wid], src_v)
    pltpu.sync_copy(dst_hbm.at[wid], dst_v)

    gbase = wid * CHUNK
    lane = lax.broadcasted_iota(jnp.int32, (16,), 0)

    def block(b, carry):
        for j in range(EB // 16):
            sidx = src_v[b, pl.ds(j * 16, 16)]
            didx = dst_v[b, pl.ds(j * 16, 16)]
            av = plsc.load_gather(tab_v, [sidx + sidx])
            dv = plsc.load_gather(tab_v, [didx + didx + 1])
            e = av + dv
            e = jnp.where(e >= 0.0, e, 0.2 * e)
            w = jnp.exp(e)
            gid = gbase + b * EB + j * 16 + lane
            w = jnp.where(gid < ET, w, 0.0)
            w_v[pl.ds(b * EB + j * 16, 16)] = w
        return carry

    lax.fori_loop(0, NBLK, block, 0)
    pltpu.sync_copy(w_v, w_hbm.at[wid])


@functools.partial(
    pl.kernel,
    out_type=jax.ShapeDtypeStruct((2, NP, DA), jnp.float32),
    mesh=_SC_MESH,
    compiler_params=_SC_PARAMS,
    scratch_types=[
        pltpu.VMEM((SB, EB), jnp.int32),
        pltpu.VMEM((SB, EB), jnp.int32),
        pltpu.VMEM((SB * EB,), jnp.float32),
        pltpu.VMEM((EB, DA), jnp.float32),
        pltpu.VMEM_SHARED((NP, DA), jnp.float32),
        pltpu.SemaphoreType.DMA,
    ],
)
def _sc_scatter(haug_hbm, src_hbm, dst_hbm, w_hbm, outp_hbm,
                sidx_v, didx_v, w_v, rows_v, acc_sh, sem):
    c = lax.axis_index("c")
    s = lax.axis_index("s")
    wid = s * 2 + c

    zero16 = jnp.zeros((16,), jnp.float32)

    def zrow(r, carry):
        for k in range(DA // 16):
            rows_v[r, pl.ds(k * 16, 16)] = zero16
        return carry

    lax.fori_loop(0, EB, zrow, 0)
    for i in range(RPT // EB):
        pltpu.sync_copy(rows_v, acc_sh.at[pl.ds(s * RPT + i * EB, EB)])
    plsc.subcore_barrier()

    for sb in range(NBLK // SB):
        pltpu.sync_copy(src_hbm.at[wid, pl.ds(sb * SB, SB)], sidx_v)
        pltpu.sync_copy(dst_hbm.at[wid, pl.ds(sb * SB, SB)], didx_v)
        pltpu.sync_copy(w_hbm.at[wid, pl.ds(sb * SB * EB, SB * EB)], w_v)

        def block(b, carry):
            pltpu.async_copy(haug_hbm.at[sidx_v.at[b]], rows_v, sem).wait()

            def scale(r, rc):
                ridx = jnp.full((16,), 0, jnp.int32) + (b * EB + r)
                wr = plsc.load_gather(w_v, [ridx])
                for k in range(DA // 16):
                    rows_v[r, pl.ds(k * 16, 16)] = rows_v[r, pl.ds(k * 16, 16)] * wr
                return rc

            lax.fori_loop(0, EB, scale, 0)
            pltpu.sync_copy(rows_v, acc_sh.at[didx_v.at[b]], add=True)
            return carry

        lax.fori_loop(0, SB, block, 0)

    plsc.subcore_barrier()
    pltpu.sync_copy(acc_sh.at[pl.ds(s * RPT, RPT)],
                    outp_hbm.at[c, pl.ds(s * RPT, RPT)])


def kernel(x, edge_index, W1, a_src1, a_dst1, b1, gamma, beta, W2, a_src2, a_dst2, b2):
    loop = jnp.arange(N, dtype=edge_index.dtype)
    padz = jnp.zeros((ET_PAD - ET,), edge_index.dtype)
    src = jnp.concatenate([edge_index[0], loop, padz]).reshape(NW, NBLK, EB)
    dst = jnp.concatenate([edge_index[1], loop, padz]).reshape(NW, NBLK, EB)
    A1 = jnp.stack([a_src1, a_dst1], axis=1)
    A2 = jnp.stack([a_src2, a_dst2], axis=1)

    haug1, asad1 = pl.pallas_call(
        _dense_pre,
        out_shape=[jax.ShapeDtypeStruct((N, DA), jnp.float32),
                   jax.ShapeDtypeStruct((N, 2), jnp.float32)],
    )(x, W1, A1)

    w1 = _sc_weights(asad1.reshape(2 * N), src, dst)
    outp1 = _sc_scatter(haug1, src, dst, w1)

    haug2, asad2 = pl.pallas_call(
        _dense_mid,
        out_shape=[jax.ShapeDtypeStruct((N, DA), jnp.float32),
                   jax.ShapeDtypeStruct((N, 2), jnp.float32)],
    )(outp1, b1.reshape(1, D), gamma.reshape(1, D), beta.reshape(1, D), W2, A2)

    w2 = _sc_weights(asad2.reshape(2 * N), src, dst)
    outp2 = _sc_scatter(haug2, src, dst, w2)

    out = pl.pallas_call(
        _dense_final,
        out_shape=jax.ShapeDtypeStruct((N, D), jnp.float32),
    )(outp2, b2.reshape(1, D))
    return out


# speedup vs baseline: 26.0750x; 1.1367x over previous
import functools
import jax
import jax.numpy as jnp
from jax import lax
from jax.experimental import pallas as pl
from jax.experimental.pallas import tpu as pltpu
from jax.experimental.pallas import tpu_sc as plsc

N = 10000
E = 320000
D = 128
DA = 144
ET = E + N
NW = 32
EB = 128
NBLK = 81
CHUNK = EB * NBLK
ET_PAD = NW * CHUNK
NP = 10240
RPT = NP // 16


def _ones_col(n):
    return jnp.where(lax.broadcasted_iota(jnp.int32, (n, 16), 1) == 0, 1.0, 0.0)


def _dense_pre(x_ref, w_ref, a_ref, haug_ref, asad_ref):
    h = jnp.dot(x_ref[...], w_ref[...], preferred_element_type=jnp.float32)
    haug_ref[:, :D] = h
    haug_ref[:, D:DA] = _ones_col(N)
    asad_ref[...] = jnp.dot(h, a_ref[...], preferred_element_type=jnp.float32)


def _dense_mid(op_ref, b_ref, g_ref, be_ref, w_ref, a_ref, haug_ref, asad_ref):
    num = op_ref[0, :N, :D] + op_ref[1, :N, :D]
    denc = op_ref[0, :N, D:DA] + op_ref[1, :N, D:DA]
    den = denc[:, 0:1]
    h = num / (den + 1e-16) + b_ref[...]
    mu = jnp.mean(h, axis=0, keepdims=True)
    var = jnp.mean(h * h, axis=0, keepdims=True) - mu * mu
    h = (h - mu) / jnp.sqrt(var + 1e-5) * g_ref[...] + be_ref[...]
    h = jnp.maximum(h, 0.0)
    h2 = jnp.dot(h, w_ref[...], preferred_element_type=jnp.float32)
    haug_ref[:, :D] = h2
    haug_ref[:, D:DA] = _ones_col(N)
    asad_ref[...] = jnp.dot(h2, a_ref[...], preferred_element_type=jnp.float32)


def _dense_final(op_ref, b_ref, out_ref):
    num = op_ref[0, :N, :D] + op_ref[1, :N, :D]
    denc = op_ref[0, :N, D:DA] + op_ref[1, :N, D:DA]
    out_ref[...] = num / (denc[:, 0:1] + 1e-16) + b_ref[...]


_SC_MESH = plsc.VectorSubcoreMesh(core_axis_name="c", subcore_axis_name="s")
_SC_PARAMS = pltpu.CompilerParams(needs_layout_passes=False,
                                  use_tc_tiling_on_sc=False)
SB = 3


@functools.partial(
    pl.kernel,
    out_type=jax.ShapeDtypeStruct((NW, CHUNK), jnp.float32),
    mesh=_SC_MESH,
    compiler_params=_SC_PARAMS,
    scratch_types=[
        pltpu.VMEM((2 * N,), jnp.float32),
        pltpu.VMEM((NBLK, EB), jnp.int32),
        pltpu.VMEM((NBLK, EB), jnp.int32),
        pltpu.VMEM((CHUNK,), jnp.float32),
    ],
)
def _sc_weights(asad_hbm, src_hbm, dst_hbm, w_hbm, tab_v, src_v, dst_v, w_v):
    c = lax.axis_index("c")
    s = lax.axis_index("s")
    wid = s * 2 + c

    pltpu.sync_copy(asad_hbm, tab_v)
    pltpu.sync_copy(src_hbm.at[wid], src_v)
    pltpu.sync_copy(dst_hbm.at[wid], dst_v)

    gbase = wid * CHUNK
    lane = lax.broadcasted_iota(jnp.int32, (16,), 0)

    def block(b, carry):
        for j in range(EB // 16):
            sidx = src_v[b, pl.ds(j * 16, 16)]
            didx = dst_v[b, pl.ds(j * 16, 16)]
            av = plsc.load_gather(tab_v, [sidx + sidx])
            dv = plsc.load_gather(tab_v, [didx + didx + 1])
            e = av + dv
            e = jnp.where(e >= 0.0, e, 0.2 * e)
            w = jnp.exp(e)
            gid = gbase + b * EB + j * 16 + lane
            w = jnp.where(gid < ET, w, 0.0)
            w_v[pl.ds(b * EB + j * 16, 16)] = w
        return carry

    lax.fori_loop(0, NBLK, block, 0)
    pltpu.sync_copy(w_v, w_hbm.at[wid])


@functools.partial(
    pl.kernel,
    out_type=jax.ShapeDtypeStruct((2, NP, DA), jnp.float32),
    mesh=_SC_MESH,
    compiler_params=_SC_PARAMS,
    scratch_types=[
        pltpu.VMEM((SB, EB), jnp.int32),
        pltpu.VMEM((SB, EB), jnp.int32),
        pltpu.VMEM((SB * EB,), jnp.float32),
        pltpu.VMEM((EB, DA), jnp.float32),
        pltpu.VMEM((EB, DA), jnp.float32),
        pltpu.VMEM_SHARED((NP, DA), jnp.float32),
        pltpu.SemaphoreType.DMA,
        pltpu.SemaphoreType.DMA,
        pltpu.SemaphoreType.DMA,
        pltpu.SemaphoreType.DMA,
        pltpu.SemaphoreType.DMA,
    ],
)
def _sc_scatter(haug_hbm, src_hbm, dst_hbm, w_hbm, outp_hbm,
                sidx_v, didx_v, w_v, rows0, rows1, acc_sh,
                semg0, semg1, sems0, sems1, semst):
    c = lax.axis_index("c")
    s = lax.axis_index("s")
    wid = s * 2 + c

    zero16 = jnp.zeros((16,), jnp.float32)

    def zrow(r, carry):
        for k in range(DA // 16):
            rows0[r, pl.ds(k * 16, 16)] = zero16
        return carry

    lax.fori_loop(0, EB, zrow, 0)
    for i in range(RPT // EB):
        pltpu.sync_copy(rows0, acc_sh.at[pl.ds(s * RPT + i * EB, EB)])
    plsc.subcore_barrier()

    rowsb = [rows0, rows1]
    semg = [semg0, semg1]
    sems = [sems0, sems1]
    hg = [None, None]
    hsc = [None, None]

    def gather(b, rb):
        if hsc[rb] is not None:
            hsc[rb].wait()
            hsc[rb] = None
        hg[rb] = pltpu.async_copy(haug_hbm.at[sidx_v.at[b]], rowsb[rb], semg[rb])

    def scale(b, rb):
        rows = rowsb[rb]

        def body(r, rc):
            ridx = jnp.full((16,), 0, jnp.int32) + (b * EB + r)
            wr = plsc.load_gather(w_v, [ridx])
            for k in range(DA // 16):
                rows[r, pl.ds(k * 16, 16)] = rows[r, pl.ds(k * 16, 16)] * wr
            return rc

        lax.fori_loop(0, EB, body, 0)

    for sb in range(NBLK // SB):
        for i in (0, 1):
            if hsc[i] is not None:
                hsc[i].wait()
                hsc[i] = None
        h1 = pltpu.async_copy(src_hbm.at[wid, pl.ds(sb * SB, SB)], sidx_v, semst)
        h2 = pltpu.async_copy(dst_hbm.at[wid, pl.ds(sb * SB, SB)], didx_v, semst)
        h3 = pltpu.async_copy(w_hbm.at[wid, pl.ds(sb * SB * EB, SB * EB)], w_v, semst)
        h1.wait(); h2.wait(); h3.wait()
        for b in range(SB):
            rb = b % 2
            if b == 0:
                gather(0, 0)
            if b + 1 < SB:
                gather(b + 1, (b + 1) % 2)
            hg[rb].wait()
            scale(b, rb)
            hsc[rb] = pltpu.async_copy(rowsb[rb], acc_sh.at[didx_v.at[b]],
                                       sems[rb], add=True)

    for i in (0, 1):
        if hsc[i] is not None:
            hsc[i].wait()
            hsc[i] = None
    plsc.subcore_barrier()
    pltpu.sync_copy(acc_sh.at[pl.ds(s * RPT, RPT)],
                    outp_hbm.at[c, pl.ds(s * RPT, RPT)])


def kernel(x, edge_index, W1, a_src1, a_dst1, b1, gamma, beta, W2, a_src2, a_dst2, b2):
    loop = jnp.arange(N, dtype=edge_index.dtype)
    padz = jnp.zeros((ET_PAD - ET,), edge_index.dtype)
    src = jnp.concatenate([edge_index[0], loop, padz]).reshape(NW, NBLK, EB)
    dst = jnp.concatenate([edge_index[1], loop, padz]).reshape(NW, NBLK, EB)
    A1 = jnp.stack([a_src1, a_dst1], axis=1)
    A2 = jnp.stack([a_src2, a_dst2], axis=1)

    haug1, asad1 = pl.pallas_call(
        _dense_pre,
        out_shape=[jax.ShapeDtypeStruct((N, DA), jnp.float32),
                   jax.ShapeDtypeStruct((N, 2), jnp.float32)],
    )(x, W1, A1)

    w1 = _sc_weights(asad1.reshape(2 * N), src, dst)
    outp1 = _sc_scatter(haug1, src, dst, w1)

    haug2, asad2 = pl.pallas_call(
        _dense_mid,
        out_shape=[jax.ShapeDtypeStruct((N, DA), jnp.float32),
                   jax.ShapeDtypeStruct((N, 2), jnp.float32)],
    )(outp1, b1.reshape(1, D), gamma.reshape(1, D), beta.reshape(1, D), W2, A2)

    w2 = _sc_weights(asad2.reshape(2 * N), src, dst)
    outp2 = _sc_scatter(haug2, src, dst, w2)

    out = pl.pallas_call(
        _dense_final,
        out_shape=jax.ShapeDtypeStruct((N, D), jnp.float32),
    )(outp2, b2.reshape(1, D))
    return out
